# Initial kernel scaffold; baseline (speedup 1.0000x reference)
import jax
import jax.numpy as jnp
from jax.experimental import pallas as pl


def kernel(device_ids, pseudo_ids, attr_ids, unit_ids, values, mask, dev_table, pseudo_table, attr_table, unit_table, val_w, val_b, in_proj_w, in_proj_b, out_proj_w, out_proj_b, out_w, out_b):
    raise NotImplementedError("write your pallas kernel here")


# speedup vs baseline: 15.9591x; 15.9591x over previous
import math

import jax
import jax.numpy as jnp
from jax.experimental import pallas as pl
from jax.experimental.pallas import tpu as pltpu

_E = 128
_H = 4
_DH = 32
_OUT = 768
_T = 5
_B = 1024


def _body(ids_ref, vals_ref, mask_ref,
          d_tab_ref, p_tab_ref, a_tab_ref, u_tab_ref,
          vw_ref, vb_ref, bias_ref, w2_ref, b2_ref,
          out_ref):
    B = out_ref.shape[0]
    ids = ids_ref[0]
    vals = vals_ref[0]
    mk = mask_ref[0].astype(jnp.float32)
    bias = bias_ref[...]

    def onehot(row, nrows):
        io = jax.lax.broadcasted_iota(jnp.int32, (nrows, B), 0)
        return jnp.where(io == row, 1.0, 0.0)

    tok = [None] * _T
    for ref, idrow, mrow in ((d_tab_ref, 0, 0), (p_tab_ref, 1, 1),
                             (a_tab_ref, 2, 2), (u_tab_ref, 3, 4)):
        tab = ref[...]
        oh = onehot(ids[idrow:idrow + 1], tab.shape[1])
        raw = jnp.dot(tab, oh, preferred_element_type=jnp.float32)
        tok[mrow] = raw * mk[mrow:mrow + 1] + bias
    raw_v = vw_ref[...] * vals + vb_ref[...]
    tok[3] = raw_v * mk[3:4] + bias

    q = [t[0:_E] for t in tok]
    k = [t[_E:2 * _E] for t in tok]
    v = [t[2 * _E:3 * _E] for t in tok]

    scale = 1.0 / math.sqrt(_DH)
    logits = [[[None] * _T for _ in range(_H)] for _ in range(_T)]
    for tq in range(_T):
        for tk in range(_T):
            prod = q[tq] * k[tk]
            for h in range(_H):
                logits[tq][h][tk] = jnp.sum(
                    prod[h * _DH:(h + 1) * _DH], axis=0, keepdims=True) * scale

    wm = [[None] * _T for _ in range(_H)]
    for tq in range(_T):
        for h in range(_H):
            l = logits[tq][h]
            m = jnp.maximum(jnp.maximum(jnp.maximum(l[0], l[1]),
                                        jnp.maximum(l[2], l[3])), l[4])
            e = [jnp.exp(x - m) for x in l]
            r = 1.0 / (e[0] + e[1] + e[2] + e[3] + e[4])
            for tk in range(_T):
                w = e[tk] * r
                wm[h][tk] = w if wm[h][tk] is None else wm[h][tk] + w

    acc = None
    for tk in range(_T):
        wfull = jnp.concatenate(
            [jnp.broadcast_to(wm[h][tk] * (1.0 / _T), (_DH, B))
             for h in range(_H)], axis=0)
        contrib = wfull * v[tk]
        acc = contrib if acc is None else acc + contrib

    outv = jax.lax.dot_general(
        acc, w2_ref[...],
        dimension_numbers=(((0,), (0,)), ((), ())),
        preferred_element_type=jnp.float32)
    out_ref[...] = outv + b2_ref[...]


def kernel(device_ids, pseudo_ids, attr_ids, unit_ids, values, mask,
           dev_table, pseudo_table, attr_table, unit_table,
           val_w, val_b, in_proj_w, in_proj_b, out_proj_w, out_proj_b,
           out_w, out_b):
    n = device_ids.shape[0]
    B = _B
    G = n // B
    f32 = jnp.float32

    ids3 = jnp.stack([device_ids, pseudo_ids, attr_ids, unit_ids]
                     ).astype(jnp.int32).reshape(4, G, B).transpose(1, 0, 2)
    vals3 = values.astype(f32).reshape(G, 1, B)
    mask3 = mask.astype(jnp.int32).T.reshape(5, G, B).transpose(1, 0, 2)

    W = in_proj_w.astype(f32)
    d_tab = W @ dev_table.T.astype(f32)
    p_tab = W @ pseudo_table.T.astype(f32)
    a_tab = W @ attr_table.T.astype(f32)
    u_tab = W @ unit_table.T.astype(f32)
    vw = jnp.broadcast_to((W @ val_w[:, 0])[:, None], (3 * _E, B))
    vb = jnp.broadcast_to((W @ val_b)[:, None], (3 * _E, B))
    bias = jnp.broadcast_to(in_proj_b[:, None], (3 * _E, B))
    w2 = out_proj_w.T.astype(f32) @ out_w.T.astype(f32)
    b2 = (out_proj_b @ out_w.T + out_b)[None, :]

    return pl.pallas_call(
        _body,
        grid=(G,),
        in_specs=[
            pl.BlockSpec((1, 4, B), lambda i: (i, 0, 0)),
            pl.BlockSpec((1, 1, B), lambda i: (i, 0, 0)),
            pl.BlockSpec((1, 5, B), lambda i: (i, 0, 0)),
            pl.BlockSpec(d_tab.shape, lambda i: (0, 0)),
            pl.BlockSpec(p_tab.shape, lambda i: (0, 0)),
            pl.BlockSpec(a_tab.shape, lambda i: (0, 0)),
            pl.BlockSpec(u_tab.shape, lambda i: (0, 0)),
            pl.BlockSpec((3 * _E, B), lambda i: (0, 0)),
            pl.BlockSpec((3 * _E, B), lambda i: (0, 0)),
            pl.BlockSpec((3 * _E, B), lambda i: (0, 0)),
            pl.BlockSpec((_E, _OUT), lambda i: (0, 0)),
            pl.BlockSpec((1, _OUT), lambda i: (0, 0)),
        ],
        out_specs=pl.BlockSpec((B, _OUT), lambda i: (i, 0)),
        out_shape=jax.ShapeDtypeStruct((n, _OUT), f32),
        compiler_params=pltpu.CompilerParams(
            dimension_semantics=("parallel",),
            vmem_limit_bytes=48 * 1024 * 1024,
        ),
    )(ids3, vals3, mask3, d_tab, p_tab, a_tab, u_tab, vw, vb, bias, w2, b2)


# speedup vs baseline: 32.2139x; 2.0185x over previous
import math

import jax
import jax.numpy as jnp
import numpy as np
from jax.experimental import pallas as pl
from jax.experimental.pallas import tpu as pltpu

_E = 128
_H = 4
_DH = 32
_OUT = 768
_B = 1024
_SCALE = 1.0 / math.sqrt(_DH)

_OFF_RV = 0
_OFF_B = 128
_OFF_SELF = 136
_OFF_TV1 = 144
_OFF_TV0 = 152
_OFF_VT1 = 160
_OFF_VT0 = 168
_OFF_PAIR = 176

_PAIRS = (
    ('p', 'd', 176, 216),
    ('a', 'd', 176, 216),
    ('a', 'p', 256, 296),
    ('a', 'u', 336, 416),
    ('u', 'd', 176, 216),
    ('u', 'p', 256, 296),
)
_TOKS = ('d', 'p', 'a', 'v', 'u')
_IDROW = {'d': 0, 'p': 1, 'a': 2, 'u': 3}
_VOCAB = {'d': 10, 'p': 10, 'a': 100, 'u': 20}
_PERM = np.arange(128)
_PERM = (_PERM % 4) * 32 + _PERM // 4


def _hd(a, b):
    p = a * b
    return p.reshape(*p.shape[:-1], _H, _DH).sum(-1) * _SCALE


def _il8(x):
    xt = jnp.moveaxis(x, -1, 0)
    return jnp.concatenate([xt, xt], axis=0)


def _pair_block(rq, rk):
    ps = _hd(rq[:, None, :], rk[None, :, :])
    return ps.transpose(0, 2, 1).reshape(rq.shape[0] * _H, rk.shape[0])


def kernel(device_ids, pseudo_ids, attr_ids, unit_ids, values, mask,
           dev_table, pseudo_table, attr_table, unit_table,
           val_w, val_b, in_proj_w, in_proj_b, out_proj_w, out_proj_b,
           out_w, out_b):
    n = device_ids.shape[0]
    B = _B
    G = n // B
    f32 = jnp.float32

    ids3 = jnp.stack([device_ids, pseudo_ids, attr_ids, unit_ids]
                     ).astype(jnp.int32).reshape(4, G, B).transpose(1, 0, 2)
    vals3 = values.astype(f32).reshape(G, 1, B)
    mask3 = mask.astype(jnp.int32).T.reshape(5, G, B).transpose(1, 0, 2)

    W = in_proj_w.astype(f32)
    Wq, Wk, Wv = W[:_E], W[_E:2 * _E], W[2 * _E:]
    bq, bk, bv = (in_proj_b[:_E].astype(f32), in_proj_b[_E:2 * _E].astype(f32),
                  in_proj_b[2 * _E:].astype(f32))
    tabs = {'d': dev_table, 'p': pseudo_table, 'a': attr_table, 'u': unit_table}
    rq = {t: tabs[t].astype(f32) @ Wq.T for t in tabs}
    rk = {t: tabs[t].astype(f32) @ Wk.T for t in tabs}
    rv = {t: tabs[t].astype(f32) @ Wv.T for t in tabs}
    vvec = val_w[:, 0].astype(f32)
    vb0 = val_b.astype(f32)
    vwq, vwk, vwv = Wq @ vvec, Wk @ vvec, Wv @ vvec
    vbq, vbk, vbv = Wq @ vb0, Wk @ vb0, Wv @ vb0

    blocks_for = {t: [] for t in tabs}
    for g, s, _, _ in _PAIRS:
        blocks_for[g].append(_pair_block(rq[s], rk[g]))
        blocks_for[g].append(_pair_block(rk[s], rq[g]))

    def build_table(t):
        R = _VOCAB[t]
        rows = [rv[t].T[_PERM],
                _il8(_hd(rk[t], bq)).reshape(8, R),
                _il8(_hd(rq[t], rk[t])).reshape(8, R),
                _il8(_hd(rq[t], vwk)).reshape(8, R),
                _il8(_hd(rq[t], vbk)).reshape(8, R),
                _il8(_hd(rk[t], vwq)).reshape(8, R),
                _il8(_hd(rk[t], vbq)).reshape(8, R)]
        rows += blocks_for[t]
        return jnp.concatenate(rows, axis=0)

    tab = {t: build_table(t) for t in tabs}

    vc_col = jnp.concatenate([
        _il8(_hd(vwq, vwk)),
        _il8(_hd(vwq, vbk) + _hd(vbq, vwk)),
        _il8(_hd(vbq, vbk)),
        _il8(_hd(bq, vwk)),
        _il8(_hd(bq, vbk)),
    ])
    vc = jnp.broadcast_to(vc_col[:, None], (40, B))
    vwv_b = jnp.broadcast_to(vwv[_PERM][:, None], (_E, B))
    vbv_b = jnp.broadcast_to(vbv[_PERM][:, None], (_E, B))

    w2 = out_proj_w.T.astype(f32) @ out_w.T.astype(f32)
    b2 = out_proj_b @ out_w.T + out_b + bv @ w2
    w2ext = jnp.concatenate(
        [w2[_PERM], b2[None, :], jnp.zeros((7, _OUT), f32)], axis=0)

    def _body(ids_ref, vals_ref, mask_ref,
              td_ref, tp_ref, ta_ref, tu_ref,
              vc_ref, vwv_ref, vbv_ref, w2_ref, out_ref):
        ids = ids_ref[0]
        vals = vals_ref[0]
        mk = mask_ref[0].astype(f32)
        trefs = {'d': td_ref, 'p': tp_ref, 'a': ta_ref, 'u': tu_ref}

        g = {}
        for t in ('d', 'p', 'a', 'u'):
            R = _VOCAB[t]
            io = jax.lax.broadcasted_iota(jnp.int32, (R, B), 0)
            oh = jnp.where(io == ids[_IDROW[t]:_IDROW[t] + 1], 1.0, 0.0)
            g[t] = jnp.dot(trefs[t][...], oh, preferred_element_type=f32)

        ohe = {}
        for t in ('d', 'p', 'u'):
            R = _VOCAB[t]
            io4 = jax.lax.broadcasted_iota(jnp.int32, (_H * R, B), 0) // _H
            ohe[t] = jnp.where(io4 == ids[_IDROW[t]:_IDROW[t] + 1], 1.0, 0.0)

        mk8 = {t: jnp.broadcast_to(mk[i:i + 1], (8, B))
               for i, t in enumerate(_TOKS)}
        val8 = jnp.broadcast_to(vals, (8, B))
        vcb = vc_ref[...]

        def seg_reduce(prod):
            nchunk = prod.shape[0] // 8
            s = prod[0:8]
            for c in range(1, nchunk):
                s = s + prod[8 * c:8 * (c + 1)]
            return s + jnp.concatenate([s[4:8], s[0:4]], axis=0)

        P = {}
        B8 = {}
        for t in ('d', 'p', 'a', 'u'):
            B8[t] = g[t][_OFF_B:_OFF_B + 8]
            P[(t, t)] = g[t][_OFF_SELF:_OFF_SELF + 8]
            P[(t, 'v')] = g[t][_OFF_TV1:_OFF_TV1 + 8] * val8 \
                + g[t][_OFF_TV0:_OFF_TV0 + 8]
            P[('v', t)] = g[t][_OFF_VT1:_OFF_VT1 + 8] * val8 \
                + g[t][_OFF_VT0:_OFF_VT0 + 8]
        B8['v'] = vcb[24:32] * val8 + vcb[32:40]
        P[('v', 'v')] = (vcb[0:8] * val8 + vcb[8:16]) * val8 + vcb[16:24]
        for gt, st, off1, off2 in _PAIRS:
            w1 = _H * _VOCAB[st]
            P[(st, gt)] = seg_reduce(g[gt][off1:off1 + w1] * ohe[st])
            P[(gt, st)] = seg_reduce(g[gt][off2:off2 + w1] * ohe[st])

        wsum = {t: None for t in _TOKS}
        for tq in _TOKS:
            ls = [mk8[tk] * (mk8[tq] * P[(tq, tk)] + B8[tk]) for tk in _TOKS]
            m = jnp.maximum(jnp.maximum(jnp.maximum(ls[0], ls[1]),
                                        jnp.maximum(ls[2], ls[3])), ls[4])
            e = [jnp.exp(x - m) for x in ls]
            r = 1.0 / (e[0] + e[1] + e[2] + e[3] + e[4])
            for i, tk in enumerate(_TOKS):
                w = e[i] * r
                wsum[tk] = w if wsum[tk] is None else wsum[tk] + w

        o = None
        for i, tk in enumerate(_TOKS):
            wm = wsum[tk] * mk8[tk] * (1.0 / len(_TOKS))
            wf = jnp.concatenate([wm] * (_E // 8), axis=0)
            if tk == 'v':
                rv_tok = vwv_ref[...] * jnp.concatenate([val8] * (_E // 8),
                                                        axis=0) + vbv_ref[...]
            else:
                rv_tok = g[tk][_OFF_RV:_OFF_RV + _E]
            c = wf * rv_tok
            o = c if o is None else o + c

        acc = jnp.concatenate([o, jnp.ones((8, B), f32)], axis=0)
        out_ref[...] = jax.lax.dot_general(
            acc, w2_ref[...],
            dimension_numbers=(((0,), (0,)), ((), ())),
            preferred_element_type=f32)

    return pl.pallas_call(
        _body,
        grid=(G,),
        in_specs=[
            pl.BlockSpec((1, 4, B), lambda i: (i, 0, 0)),
            pl.BlockSpec((1, 1, B), lambda i: (i, 0, 0)),
            pl.BlockSpec((1, 5, B), lambda i: (i, 0, 0)),
            pl.BlockSpec(tab['d'].shape, lambda i: (0, 0)),
            pl.BlockSpec(tab['p'].shape, lambda i: (0, 0)),
            pl.BlockSpec(tab['a'].shape, lambda i: (0, 0)),
            pl.BlockSpec(tab['u'].shape, lambda i: (0, 0)),
            pl.BlockSpec((40, B), lambda i: (0, 0)),
            pl.BlockSpec((_E, B), lambda i: (0, 0)),
            pl.BlockSpec((_E, B), lambda i: (0, 0)),
            pl.BlockSpec((136, _OUT), lambda i: (0, 0)),
        ],
        out_specs=pl.BlockSpec((B, _OUT), lambda i: (i, 0)),
        out_shape=jax.ShapeDtypeStruct((n, _OUT), f32),
        compiler_params=pltpu.CompilerParams(
            dimension_semantics=("parallel",),
            vmem_limit_bytes=48 * 1024 * 1024,
        ),
    )(ids3, vals3, mask3, tab['d'], tab['p'], tab['a'], tab['u'],
      vc, vwv_b, vbv_b, w2ext)


# speedup vs baseline: 32.6319x; 1.0130x over previous
import math

import jax
import jax.numpy as jnp
import numpy as np
from jax.experimental import pallas as pl
from jax.experimental.pallas import tpu as pltpu

_E = 128
_H = 4
_DH = 32
_OUT = 768
_CB = 1024
_NCHUNK = 2
_B = _CB * _NCHUNK
_SCALE = 1.0 / math.sqrt(_DH)

_OFF_RV = 0
_OFF_B = 128
_OFF_SELF = 136
_OFF_TV1 = 144
_OFF_TV0 = 152
_OFF_VT1 = 160
_OFF_VT0 = 168
_OFF_PAIR = 176

_PAIRS = (
    ('p', 'd', 176, 216),
    ('a', 'd', 176, 216),
    ('a', 'p', 256, 296),
    ('a', 'u', 336, 416),
    ('u', 'd', 176, 216),
    ('u', 'p', 256, 296),
)
_TOKS = ('d', 'p', 'a', 'v', 'u')
_VOCAB = {'d': 10, 'p': 10, 'a': 100, 'u': 20}
_PERM = np.arange(128)
_PERM = (_PERM % 4) * 32 + _PERM // 4


def _hd(a, b):
    p = a * b
    return p.reshape(*p.shape[:-1], _H, _DH).sum(-1) * _SCALE


def _il8(x):
    xt = jnp.moveaxis(x, -1, 0)
    return jnp.concatenate([xt, xt], axis=0)


def _pair_block(rq, rk):
    ps = _hd(rq[:, None, :], rk[None, :, :])
    return ps.transpose(0, 2, 1).reshape(rq.shape[0] * _H, rk.shape[0])


def kernel(device_ids, pseudo_ids, attr_ids, unit_ids, values, mask,
           dev_table, pseudo_table, attr_table, unit_table,
           val_w, val_b, in_proj_w, in_proj_b, out_proj_w, out_proj_b,
           out_w, out_b):
    n = device_ids.shape[0]
    B = _B
    G = n // B
    f32 = jnp.float32

    ids_in = {
        'd': device_ids.astype(jnp.int32).reshape(G, 1, B),
        'p': pseudo_ids.astype(jnp.int32).reshape(G, 1, B),
        'a': attr_ids.astype(jnp.int32).reshape(G, 1, B),
        'u': unit_ids.astype(jnp.int32).reshape(G, 1, B),
    }
    vals3 = values.astype(f32).reshape(G, 1, B)
    mask3 = mask.astype(jnp.int32).T.reshape(5, G, B).transpose(1, 0, 2)

    W = in_proj_w.astype(f32)
    Wq, Wk, Wv = W[:_E], W[_E:2 * _E], W[2 * _E:]
    bq, bk, bv = (in_proj_b[:_E].astype(f32), in_proj_b[_E:2 * _E].astype(f32),
                  in_proj_b[2 * _E:].astype(f32))
    tabs = {'d': dev_table, 'p': pseudo_table, 'a': attr_table, 'u': unit_table}
    rq = {t: tabs[t].astype(f32) @ Wq.T for t in tabs}
    rk = {t: tabs[t].astype(f32) @ Wk.T for t in tabs}
    rv = {t: tabs[t].astype(f32) @ Wv.T for t in tabs}
    vvec = val_w[:, 0].astype(f32)
    vb0 = val_b.astype(f32)
    vwq, vwk, vwv = Wq @ vvec, Wk @ vvec, Wv @ vvec
    vbq, vbk, vbv = Wq @ vb0, Wk @ vb0, Wv @ vb0

    blocks_for = {t: [] for t in tabs}
    for g, s, _, _ in _PAIRS:
        blocks_for[g].append(_pair_block(rq[s], rk[g]))
        blocks_for[g].append(_pair_block(rk[s], rq[g]))

    def build_table(t):
        R = _VOCAB[t]
        rows = [rv[t].T[_PERM],
                _il8(_hd(rk[t], bq)).reshape(8, R),
                _il8(_hd(rq[t], rk[t])).reshape(8, R),
                _il8(_hd(rq[t], vwk)).reshape(8, R),
                _il8(_hd(rq[t], vbk)).reshape(8, R),
                _il8(_hd(rk[t], vwq)).reshape(8, R),
                _il8(_hd(rk[t], vbq)).reshape(8, R)]
        rows += blocks_for[t]
        return jnp.concatenate(rows, axis=0)

    tab = {t: build_table(t) for t in tabs}

    vc_col = jnp.concatenate([
        _il8(_hd(vwq, vwk)),
        _il8(_hd(vwq, vbk) + _hd(vbq, vwk)),
        _il8(_hd(vbq, vbk)),
        _il8(_hd(bq, vwk)),
        _il8(_hd(bq, vbk)),
    ])
    vc = jnp.broadcast_to(vc_col[:, None], (40, _CB))
    vwv_b = jnp.broadcast_to(vwv[_PERM][:, None], (_E, _CB))
    vbv_b = jnp.broadcast_to(vbv[_PERM][:, None], (_E, _CB))

    w2 = out_proj_w.T.astype(f32) @ out_w.T.astype(f32)
    b2 = out_proj_b @ out_w.T + out_b + bv @ w2
    w2ext = jnp.concatenate(
        [w2[_PERM], b2[None, :], jnp.zeros((7, _OUT), f32)], axis=0)

    def _body(idd_ref, idp_ref, ida_ref, idu_ref, vals_ref, mask_ref,
              td_ref, tp_ref, ta_ref, tu_ref,
              vc_ref, vwv_ref, vbv_ref, w2_ref, out_ref):
        idrefs = {'d': idd_ref, 'p': idp_ref, 'a': ida_ref, 'u': idu_ref}
        trefs = {'d': td_ref, 'p': tp_ref, 'a': ta_ref, 'u': tu_ref}
        vcb = vc_ref[...]

        def do_chunk(c):
            sl = slice(c * _CB, (c + 1) * _CB)
            ids = {t: idrefs[t][0][:, sl] for t in idrefs}
            vals = vals_ref[0][:, sl]
            mk = mask_ref[0][:, sl].astype(f32)

            g = {}
            for t in ('d', 'p', 'a', 'u'):
                R = _VOCAB[t]
                io = jax.lax.broadcasted_iota(jnp.int32, (R, _CB), 0)
                oh = jnp.where(io == ids[t], 1.0, 0.0)
                g[t] = jnp.dot(trefs[t][...], oh, preferred_element_type=f32)

            ohe = {}
            for t in ('d', 'p', 'u'):
                R = _VOCAB[t]
                io4 = jax.lax.broadcasted_iota(
                    jnp.int32, (_H * R, _CB), 0) // _H
                ohe[t] = jnp.where(io4 == ids[t], 1.0, 0.0)

            mk8 = {t: jnp.broadcast_to(mk[i:i + 1], (8, _CB))
                   for i, t in enumerate(_TOKS)}
            val8 = jnp.broadcast_to(vals, (8, _CB))

            def seg_reduce(prod):
                s = prod[0:8]
                for k in range(1, prod.shape[0] // 8):
                    s = s + prod[8 * k:8 * (k + 1)]
                return s + jnp.concatenate([s[4:8], s[0:4]], axis=0)

            P = {}
            B8 = {}
            for t in ('d', 'p', 'a', 'u'):
                B8[t] = g[t][_OFF_B:_OFF_B + 8]
                P[(t, t)] = g[t][_OFF_SELF:_OFF_SELF + 8]
                P[(t, 'v')] = g[t][_OFF_TV1:_OFF_TV1 + 8] * val8 \
                    + g[t][_OFF_TV0:_OFF_TV0 + 8]
                P[('v', t)] = g[t][_OFF_VT1:_OFF_VT1 + 8] * val8 \
                    + g[t][_OFF_VT0:_OFF_VT0 + 8]
            B8['v'] = vcb[24:32] * val8 + vcb[32:40]
            P[('v', 'v')] = (vcb[0:8] * val8 + vcb[8:16]) * val8 + vcb[16:24]
            for gt, st, off1, off2 in _PAIRS:
                w1 = _H * _VOCAB[st]
                P[(st, gt)] = seg_reduce(g[gt][off1:off1 + w1] * ohe[st])
                P[(gt, st)] = seg_reduce(g[gt][off2:off2 + w1] * ohe[st])

            wsum = {t: None for t in _TOKS}
            for tq in _TOKS:
                ls = [mk8[tk] * (mk8[tq] * P[(tq, tk)] + B8[tk])
                      for tk in _TOKS]
                m = jnp.maximum(jnp.maximum(jnp.maximum(ls[0], ls[1]),
                                            jnp.maximum(ls[2], ls[3])), ls[4])
                e = [jnp.exp(x - m) for x in ls]
                r = 1.0 / (e[0] + e[1] + e[2] + e[3] + e[4])
                for i, tk in enumerate(_TOKS):
                    w = e[i] * r
                    wsum[tk] = w if wsum[tk] is None else wsum[tk] + w

            o = None
            for tk in _TOKS:
                wm = wsum[tk] * mk8[tk] * (1.0 / len(_TOKS))
                wf = jnp.concatenate([wm] * (_E // 8), axis=0)
                if tk == 'v':
                    rv_tok = vwv_ref[...] * jnp.concatenate(
                        [val8] * (_E // 8), axis=0) + vbv_ref[...]
                else:
                    rv_tok = g[tk][_OFF_RV:_OFF_RV + _E]
                ctr = wf * rv_tok
                o = ctr if o is None else o + ctr

            acc = jnp.concatenate([o, jnp.ones((8, _CB), f32)],
                                  axis=0)
            out_ref[sl, :] = jax.lax.dot_general(
                acc, w2_ref[...],
                dimension_numbers=(((0,), (0,)), ((), ())),
                preferred_element_type=f32)

        for c in range(_NCHUNK):
            do_chunk(c)

    return pl.pallas_call(
        _body,
        grid=(G,),
        in_specs=[
            pl.BlockSpec((1, 1, B), lambda i: (i, 0, 0)),
            pl.BlockSpec((1, 1, B), lambda i: (i, 0, 0)),
            pl.BlockSpec((1, 1, B), lambda i: (i, 0, 0)),
            pl.BlockSpec((1, 1, B), lambda i: (i, 0, 0)),
            pl.BlockSpec((1, 1, B), lambda i: (i, 0, 0)),
            pl.BlockSpec((1, 5, B), lambda i: (i, 0, 0)),
            pl.BlockSpec(tab['d'].shape, lambda i: (0, 0)),
            pl.BlockSpec(tab['p'].shape, lambda i: (0, 0)),
            pl.BlockSpec(tab['a'].shape, lambda i: (0, 0)),
            pl.BlockSpec(tab['u'].shape, lambda i: (0, 0)),
            pl.BlockSpec((40, _CB), lambda i: (0, 0)),
            pl.BlockSpec((_E, _CB), lambda i: (0, 0)),
            pl.BlockSpec((_E, _CB), lambda i: (0, 0)),
            pl.BlockSpec((136, _OUT), lambda i: (0, 0)),
        ],
        out_specs=pl.BlockSpec((B, _OUT), lambda i: (i, 0)),
        out_shape=jax.ShapeDtypeStruct((n, _OUT), f32),
        compiler_params=pltpu.CompilerParams(
            dimension_semantics=("parallel",),
            vmem_limit_bytes=56 * 1024 * 1024,
        ),
    )(ids_in['d'], ids_in['p'], ids_in['a'], ids_in['u'], vals3, mask3,
      tab['d'], tab['p'], tab['a'], tab['u'], vc, vwv_b, vbv_b, w2ext)
